# baseline (speedup 1.0000x reference)
import functools

import jax
import jax.numpy as jnp
from jax import lax
from jax.experimental import pallas as pl
from jax.experimental.pallas import tpu as pltpu
from jax.experimental.pallas import tpu_sc as plsc

N, D, B, DEG_VOCAB = 100000, 128, 16, 65
L = 16
C = 128
NC, NS = 2, 16
NW = NC * NS
NT_FULL = N // C
TAIL = N - NT_FULL * C
TAIL_BASE = NT_FULL * C
TPW = NT_FULL // NW
XTRA = NT_FULL - TPW * NW
IDXC = (TPW + 1) * C
B2_CHUNK = 3136
B2_LAST = (N + B) - (NW - 1) * B2_CHUNK


def _sc_encode(x_hbm, din_hbm, dout_hbm, starts_hbm, tin_hbm, tout_hbm,
               gt_hbm, outh_hbm, outb_hbm,
               starts_v, xh0, xh1, xh2, tin_v, tout_v,
               off0, off1, off2,
               dina, douta, gt_v, tok_v, tokidx_v, b2_v,
               dint_v, doutt_v, offt_v,
               semx0, semx1, semx2,
               sems0, sems1, sems2, semi):
    wid = lax.axis_index("s") * NC + lax.axis_index("c")
    iota = lax.iota(jnp.int32, L)

    pltpu.sync_copy(starts_hbm, starts_v)
    _sv = starts_v[...]
    splats = [_sv.at[jnp.full((L,), g, jnp.int32)].get(mode="promise_in_bounds")
              for g in range(B)]
    ones = jnp.full((L,), 1, jnp.int32)
    zeros = jnp.full((L,), 0, jnp.int32)

    def count_starts_le(i_v):
        cnt = jnp.zeros((L,), jnp.int32)
        for g in range(B):
            cnt = cnt + jnp.where(i_v >= splats[g], ones, zeros)
        return cnt

    t0 = wid * TPW + jnp.minimum(wid, XTRA)
    nt = TPW + jnp.minimum(jnp.maximum(XTRA - wid, 0), 1)
    base_row = t0 * C

    @pl.when(wid < XTRA)
    def _():
        pltpu.async_copy(din_hbm.at[pl.ds(base_row, IDXC)], dina, semi).wait()
        pltpu.async_copy(dout_hbm.at[pl.ds(base_row, IDXC)], douta, semi).wait()

    @pl.when(wid >= XTRA)
    def _():
        pltpu.async_copy(din_hbm.at[pl.ds(base_row, TPW * C)],
                         dina.at[pl.ds(0, TPW * C)], semi).wait()
        pltpu.async_copy(dout_hbm.at[pl.ds(base_row, TPW * C)],
                         douta.at[pl.ds(0, TPW * C)], semi).wait()

    pltpu.sync_copy(tin_hbm, tin_v)
    pltpu.sync_copy(tout_hbm, tout_v)
    lane_consts = [jnp.full((L,), l, jnp.int32) for l in range(L)]
    col_consts = [c8 * L + iota for c8 in range(D // L)]

    xbufs = (xh0, xh1, xh2)
    offs = (off0, off1, off2)
    xsems = (semx0, semx1, semx2)
    ssems = (sems0, sems1, sems2)

    def x_copy(k, b3):
        return pltpu.make_async_copy(
            x_hbm.at[pl.ds(base_row + k * C, C)], xbufs[b3], xsems[b3])

    def scatter_copy(b3):
        return pltpu.make_async_copy(xbufs[b3], outh_hbm.at[offs[b3]],
                                     ssems[b3])

    def add_embeddings(xh, din_ref, dout_ref, dbase, nrows):
        @plsc.parallel_loop(0, nrows // L, unroll=2)
        def grp(g16):
            din16 = din_ref[pl.ds(dbase + g16 * L, L)] * D
            dout16 = dout_ref[pl.ds(dbase + g16 * L, L)] * D
            for l in range(L):
                sin = din16.at[lane_consts[l]].get(mode="promise_in_bounds")
                sout = dout16.at[lane_consts[l]].get(mode="promise_in_bounds")
                r = g16 * L + l
                for c8 in range(D // L):
                    s = pl.ds(c8 * L, L)
                    xh[r, s] = (xh[r, s]
                                + plsc.load_gather(tin_v, [sin + col_consts[c8]])
                                + plsc.load_gather(tout_v, [sout + col_consts[c8]]))

    def compute(k, b3):
        xh, off = xbufs[b3], offs[b3]
        base = base_row + k * C

        @plsc.parallel_loop(0, C // L)
        def og(j):
            i_v = base + j * L + iota
            off[pl.ds(j * L, L)] = i_v + count_starts_le(i_v)
        add_embeddings(xh, dina, douta, k * C, C)

    x_copy(0, 0).start()

    def tri_body(p, _):
        for b3 in range(3):
            k = 3 * p + b3

            @pl.when(k < nt)
            def _():
                x_copy(k, b3).wait()

                @pl.when(k + 1 < nt)
                def _():
                    @pl.when(k >= 2)
                    def _():
                        scatter_copy((b3 + 1) % 3).wait()
                    x_copy(k + 1, (b3 + 1) % 3).start()
                compute(k, b3)
                scatter_copy(b3).start()
        return 0

    lax.fori_loop(0, (TPW + 3) // 3, tri_body, 0)
    scatter_copy(0).wait()
    scatter_copy(1).wait()
    scatter_copy(2).wait()

    @pl.when(wid == 0)
    def _():
        pltpu.sync_copy(x_hbm.at[pl.ds(TAIL_BASE, TAIL)], xh0.at[pl.ds(0, TAIL)])
        pltpu.sync_copy(din_hbm.at[pl.ds(TAIL_BASE, TAIL)], dint_v)
        pltpu.sync_copy(dout_hbm.at[pl.ds(TAIL_BASE, TAIL)], doutt_v)

        def ogt(j, _):
            i_v = TAIL_BASE + j * L + iota
            offt_v[pl.ds(j * L, L)] = i_v + count_starts_le(i_v)
            return 0
        lax.fori_loop(0, TAIL // L, ogt, 0)
        add_embeddings(xh0, dint_v, doutt_v, 0, TAIL)
        pltpu.async_copy(xh0.at[pl.ds(0, TAIL)], outh_hbm.at[offt_v], semi).wait()

        pltpu.sync_copy(gt_hbm, gt_v)
        tokidx_v[...] = starts_v[...] + iota

        def trow(r, _):
            for c8 in range(D // L):
                s = pl.ds(c8 * L, L)
                tok_v[r, s] = gt_v[0, s]
            return 0
        lax.fori_loop(0, B, trow, 0)
        pltpu.async_copy(tok_v, outh_hbm.at[tokidx_v], semi).wait()

    toks = [splats[g] + g for g in range(B)]
    b2base = wid * B2_CHUNK

    @plsc.parallel_loop(0, B2_CHUNK // L)
    def bg(j):
        j_v = b2base + j * L + iota
        cnt = jnp.zeros((L,), jnp.int32)
        for g in range(B):
            cnt = cnt + jnp.where(j_v >= toks[g], ones, zeros)
        b2_v[pl.ds(j * L, L)] = cnt - 1

    @pl.when(wid < NW - 1)
    def _():
        pltpu.sync_copy(b2_v, outb_hbm.at[pl.ds(b2base, B2_CHUNK)])

    @pl.when(wid == NW - 1)
    def _():
        pltpu.sync_copy(b2_v.at[pl.ds(0, B2_LAST)],
                        outb_hbm.at[pl.ds(b2base, B2_LAST)])


_sc_call = functools.partial(
    pl.kernel,
    mesh=plsc.VectorSubcoreMesh(core_axis_name="c", subcore_axis_name="s"),
    compiler_params=pltpu.CompilerParams(needs_layout_passes=False),
    out_type=[
        jax.ShapeDtypeStruct((N + B, D), jnp.float32),
        jax.ShapeDtypeStruct((N + B,), jnp.int32),
    ],
    scratch_types=[
        pltpu.VMEM((B,), jnp.int32),
        pltpu.VMEM((C, D), jnp.float32),
        pltpu.VMEM((C, D), jnp.float32),
        pltpu.VMEM((C, D), jnp.float32),
        pltpu.VMEM((DEG_VOCAB * D,), jnp.float32),
        pltpu.VMEM((DEG_VOCAB * D,), jnp.float32),
        pltpu.VMEM((C,), jnp.int32),
        pltpu.VMEM((C,), jnp.int32),
        pltpu.VMEM((C,), jnp.int32),
        pltpu.VMEM((IDXC,), jnp.int32),
        pltpu.VMEM((IDXC,), jnp.int32),
        pltpu.VMEM((1, D), jnp.float32),
        pltpu.VMEM((B, D), jnp.float32),
        pltpu.VMEM((B,), jnp.int32),
        pltpu.VMEM((B2_CHUNK,), jnp.int32),
        pltpu.VMEM((TAIL,), jnp.int32),
        pltpu.VMEM((TAIL,), jnp.int32),
        pltpu.VMEM((TAIL,), jnp.int32),
        pltpu.SemaphoreType.DMA,
        pltpu.SemaphoreType.DMA,
        pltpu.SemaphoreType.DMA,
        pltpu.SemaphoreType.DMA,
        pltpu.SemaphoreType.DMA,
        pltpu.SemaphoreType.DMA,
        pltpu.SemaphoreType.DMA,
    ],
)(_sc_encode)


def kernel(x, in_degree, out_degree, batch, in_table, out_table, graph_token):
    starts = jnp.searchsorted(
        batch, jnp.arange(B, dtype=batch.dtype)).astype(jnp.int32)
    out_h, out_b2 = _sc_call(
        x, in_degree.astype(jnp.int32), out_degree.astype(jnp.int32),
        starts, in_table.reshape(-1), out_table.reshape(-1), graph_token)
    return out_h, out_b2


# speedup vs baseline: 1.2362x; 1.2362x over previous
import functools

import jax
import jax.numpy as jnp
from jax import lax
from jax.experimental import pallas as pl
from jax.experimental.pallas import tpu as pltpu
from jax.experimental.pallas import tpu_sc as plsc

N, D, B, DEG_VOCAB = 100000, 128, 16, 65
L = 16
C = 128
NC, NS = 2, 16
NW = NC * NS
NT_FULL = N // C
TAIL = N - NT_FULL * C
TAIL_BASE = NT_FULL * C
TPW = NT_FULL // NW
XTRA = NT_FULL - TPW * NW
IDXC = (TPW + 1) * C
B2_CHUNK = 3136
B2_LAST = (N + B) - (NW - 1) * B2_CHUNK


def _sc_encode(x_hbm, din_hbm, dout_hbm, starts_hbm, tin_hbm, tout_hbm,
               gt_hbm, outh_hbm, outb_hbm,
               starts_v, xh0, xh1, xh2, tin_v, tout_v,
               off0, off1, off2,
               dina, douta, gt_v, tok_v, tokidx_v, b2_v,
               dint_v, doutt_v, offt_v,
               semx0, semx1, semx2,
               sems0, sems1, sems2, semi):
    wid = lax.axis_index("s") * NC + lax.axis_index("c")
    iota = lax.iota(jnp.int32, L)

    pltpu.sync_copy(starts_hbm, starts_v)
    _sv = starts_v[...]
    splats = [_sv.at[jnp.full((L,), g, jnp.int32)].get(mode="promise_in_bounds")
              for g in range(B)]
    ones = jnp.full((L,), 1, jnp.int32)
    zeros = jnp.full((L,), 0, jnp.int32)

    def count_starts_le(i_v):
        cnt = jnp.zeros((L,), jnp.int32)
        for g in range(B):
            cnt = cnt + jnp.where(i_v >= splats[g], ones, zeros)
        return cnt

    t0 = wid * TPW + jnp.minimum(wid, XTRA)
    nt = TPW + jnp.minimum(jnp.maximum(XTRA - wid, 0), 1)
    base_row = t0 * C

    @pl.when(wid < XTRA)
    def _():
        pltpu.async_copy(din_hbm.at[pl.ds(base_row, IDXC)], dina, semi).wait()
        pltpu.async_copy(dout_hbm.at[pl.ds(base_row, IDXC)], douta, semi).wait()

    @pl.when(wid >= XTRA)
    def _():
        pltpu.async_copy(din_hbm.at[pl.ds(base_row, TPW * C)],
                         dina.at[pl.ds(0, TPW * C)], semi).wait()
        pltpu.async_copy(dout_hbm.at[pl.ds(base_row, TPW * C)],
                         douta.at[pl.ds(0, TPW * C)], semi).wait()

    pltpu.sync_copy(tin_hbm, tin_v)
    pltpu.sync_copy(tout_hbm, tout_v)
    lane_consts = [jnp.full((L,), l, jnp.int32) for l in range(L)]
    col_consts = [c8 * L + iota for c8 in range(D // L)]

    xbufs = (xh0, xh1, xh2)
    offs = (off0, off1, off2)
    xsems = (semx0, semx1, semx2)
    ssems = (sems0, sems1, sems2)

    def x_copy(k, b3):
        return pltpu.make_async_copy(
            x_hbm.at[pl.ds(base_row + k * C, C)], xbufs[b3], xsems[b3])

    def scatter_copy(b3):
        return pltpu.make_async_copy(xbufs[b3], outh_hbm.at[offs[b3]],
                                     ssems[b3])

    def add_embeddings(xh, din_ref, dout_ref, dbase, nrows):
        @plsc.parallel_loop(0, nrows // L)
        def grp(g16):
            din16 = din_ref[pl.ds(dbase + g16 * L, L)] * D
            dout16 = dout_ref[pl.ds(dbase + g16 * L, L)] * D
            for l in range(L):
                sin = din16.at[lane_consts[l]].get(mode="promise_in_bounds")
                sout = dout16.at[lane_consts[l]].get(mode="promise_in_bounds")
                r = g16 * L + l
                for c8 in range(D // L):
                    s = pl.ds(c8 * L, L)
                    xh[r, s] = (xh[r, s]
                                + plsc.load_gather(tin_v, [sin + col_consts[c8]])
                                + plsc.load_gather(tout_v, [sout + col_consts[c8]]))

    def compute(k, b3):
        xh, off = xbufs[b3], offs[b3]
        base = base_row + k * C

        @plsc.parallel_loop(0, C // L)
        def og(j):
            i_v = base + j * L + iota
            off[pl.ds(j * L, L)] = i_v + count_starts_le(i_v)
        add_embeddings(xh, dina, douta, k * C, C)

    x_copy(0, 0).start()

    def tri_body(p, _):
        for b3 in range(3):
            k = 3 * p + b3

            @pl.when(k < nt)
            def _():
                x_copy(k, b3).wait()

                @pl.when(k + 1 < nt)
                def _():
                    @pl.when(k >= 2)
                    def _():
                        scatter_copy((b3 + 1) % 3).wait()
                    x_copy(k + 1, (b3 + 1) % 3).start()
                compute(k, b3)
                scatter_copy(b3).start()
        return 0

    lax.fori_loop(0, (TPW + 3) // 3, tri_body, 0)
    scatter_copy(0).wait()
    scatter_copy(1).wait()
    scatter_copy(2).wait()

    @pl.when(wid == 0)
    def _():
        pltpu.sync_copy(x_hbm.at[pl.ds(TAIL_BASE, TAIL)], xh0.at[pl.ds(0, TAIL)])
        pltpu.sync_copy(din_hbm.at[pl.ds(TAIL_BASE, TAIL)], dint_v)
        pltpu.sync_copy(dout_hbm.at[pl.ds(TAIL_BASE, TAIL)], doutt_v)

        def ogt(j, _):
            i_v = TAIL_BASE + j * L + iota
            offt_v[pl.ds(j * L, L)] = i_v + count_starts_le(i_v)
            return 0
        lax.fori_loop(0, TAIL // L, ogt, 0)
        add_embeddings(xh0, dint_v, doutt_v, 0, TAIL)
        pltpu.async_copy(xh0.at[pl.ds(0, TAIL)], outh_hbm.at[offt_v], semi).wait()

        pltpu.sync_copy(gt_hbm, gt_v)
        tokidx_v[...] = starts_v[...] + iota

        def trow(r, _):
            for c8 in range(D // L):
                s = pl.ds(c8 * L, L)
                tok_v[r, s] = gt_v[0, s]
            return 0
        lax.fori_loop(0, B, trow, 0)
        pltpu.async_copy(tok_v, outh_hbm.at[tokidx_v], semi).wait()

    toks = [splats[g] + g for g in range(B)]
    b2base = wid * B2_CHUNK

    @plsc.parallel_loop(0, B2_CHUNK // L)
    def bg(j):
        j_v = b2base + j * L + iota
        cnt = jnp.zeros((L,), jnp.int32)
        for g in range(B):
            cnt = cnt + jnp.where(j_v >= toks[g], ones, zeros)
        b2_v[pl.ds(j * L, L)] = cnt - 1

    @pl.when(wid < NW - 1)
    def _():
        pltpu.sync_copy(b2_v, outb_hbm.at[pl.ds(b2base, B2_CHUNK)])

    @pl.when(wid == NW - 1)
    def _():
        pltpu.sync_copy(b2_v.at[pl.ds(0, B2_LAST)],
                        outb_hbm.at[pl.ds(b2base, B2_LAST)])


_sc_call = functools.partial(
    pl.kernel,
    mesh=plsc.VectorSubcoreMesh(core_axis_name="c", subcore_axis_name="s"),
    compiler_params=pltpu.CompilerParams(needs_layout_passes=False),
    out_type=[
        jax.ShapeDtypeStruct((N + B, D), jnp.float32),
        jax.ShapeDtypeStruct((N + B,), jnp.int32),
    ],
    scratch_types=[
        pltpu.VMEM((B,), jnp.int32),
        pltpu.VMEM((C, D), jnp.float32),
        pltpu.VMEM((C, D), jnp.float32),
        pltpu.VMEM((C, D), jnp.float32),
        pltpu.VMEM((DEG_VOCAB * D,), jnp.float32),
        pltpu.VMEM((DEG_VOCAB * D,), jnp.float32),
        pltpu.VMEM((C,), jnp.int32),
        pltpu.VMEM((C,), jnp.int32),
        pltpu.VMEM((C,), jnp.int32),
        pltpu.VMEM((IDXC,), jnp.int32),
        pltpu.VMEM((IDXC,), jnp.int32),
        pltpu.VMEM((1, D), jnp.float32),
        pltpu.VMEM((B, D), jnp.float32),
        pltpu.VMEM((B,), jnp.int32),
        pltpu.VMEM((B2_CHUNK,), jnp.int32),
        pltpu.VMEM((TAIL,), jnp.int32),
        pltpu.VMEM((TAIL,), jnp.int32),
        pltpu.VMEM((TAIL,), jnp.int32),
        pltpu.SemaphoreType.DMA,
        pltpu.SemaphoreType.DMA,
        pltpu.SemaphoreType.DMA,
        pltpu.SemaphoreType.DMA,
        pltpu.SemaphoreType.DMA,
        pltpu.SemaphoreType.DMA,
        pltpu.SemaphoreType.DMA,
    ],
)(_sc_encode)


def kernel(x, in_degree, out_degree, batch, in_table, out_table, graph_token):
    starts = jnp.searchsorted(
        batch, jnp.arange(B, dtype=batch.dtype)).astype(jnp.int32)
    out_h, out_b2 = _sc_call(
        x, in_degree.astype(jnp.int32), out_degree.astype(jnp.int32),
        starts, in_table.reshape(-1), out_table.reshape(-1), graph_token)
    return out_h, out_b2
